# baseline (speedup 1.0000x reference)
import functools

import jax
import jax.numpy as jnp
from jax.experimental import pallas as pl
from jax.experimental.pallas import tpu as pltpu

N = 4096
D = 256
BLK = 128
GRID = N // BLK


def _bitonic_sort_lanes(x, n):
    lane = jax.lax.broadcasted_iota(jnp.int32, (1, n), 1)
    k = 2
    while k <= n:
        j = k // 2
        while j >= 1:
            up = pltpu.roll(x, n - j, 1)
            down = pltpu.roll(x, j, 1)
            low_half = (lane & j) == 0
            partner = jnp.where(low_half, up, down)
            want_min = low_half ^ ((lane & k) != 0)
            x = jnp.where(want_min, jnp.minimum(x, partner),
                          jnp.maximum(x, partner))
            j //= 2
        k *= 2
    return x


def _ka_body(u_ref, vs_ref, v_ref):
    s = jax.lax.dot_general(u_ref[...], vs_ref[...],
                            (((1,), (1,)), ((), ())),
                            preferred_element_type=jnp.float32)
    ssort = _bitonic_sort_lanes(s, N)
    norm2 = jnp.sum(ssort * ssort)
    v_ref[...] = (ssort / jnp.sqrt(norm2)).astype(jnp.bfloat16)


def _bitonic_sort_sublanes(x, n):
    row = jax.lax.broadcasted_iota(jnp.int32, (n, 1), 0)
    strides = [1 << b for b in range(12)]
    jmask = {j: (row & j) == 0 for j in strides}
    kmask = {k: (row & k) != 0 for k in [s * 2 for s in strides]}
    k = 2
    while k <= n:
        j = k // 2
        while j >= 1:
            up = pltpu.roll(x, n - j, 0)
            down = pltpu.roll(x, j, 0)
            low_half = jmask[j]
            partner = jnp.where(low_half, up, down)
            want_min = low_half ^ kmask[k]
            x = jnp.where(want_min, jnp.minimum(x, partner),
                          jnp.maximum(x, partner))
            j //= 2
        k *= 2
    return x


def _kb_body(v_ref, t_ref, tb_ref, row_ref, idx_ref, maxv_sc, maxi_sc):
    jidx = pl.program_id(0)
    g = jax.lax.dot_general(t_ref[...], tb_ref[...],
                            (((1,), (1,)), ((), ())),
                            preferred_element_type=jnp.float32)
    gs = _bitonic_sort_sublanes(g, N)
    norms2 = jnp.sum(gs * gs, axis=0, keepdims=True)
    gsn = (gs / jnp.sqrt(norms2)).astype(jnp.bfloat16)
    vals = jax.lax.dot_general(v_ref[...], gsn,
                               (((1,), (0,)), ((), ())),
                               preferred_element_type=jnp.float32)
    row_ref[...] = vals

    m = jnp.max(vals)
    lane = jax.lax.broadcasted_iota(jnp.int32, (1, BLK), 1) + jidx * BLK
    li = jnp.min(jnp.where(vals == m, lane, jnp.int32(2**30)))

    prev_v = maxv_sc[0]
    better = jnp.logical_or(jidx == 0, m > prev_v)
    maxv_sc[0] = jnp.where(better, m, prev_v)
    maxi_sc[0] = jnp.where(better, li, maxi_sc[0])

    @pl.when(jidx == GRID - 1)
    def _():
        idx_ref[0] = maxi_sc[0]


def kernel(vectors_source, vectors_target, src_idx):
    u = jnp.take(vectors_source, src_idx, axis=0).reshape(1, D)
    u_bf = u.astype(jnp.bfloat16)
    vs_bf = vectors_source.astype(jnp.bfloat16)
    t_bf = vectors_target.astype(jnp.bfloat16)

    v = pl.pallas_call(
        _ka_body,
        out_shape=jax.ShapeDtypeStruct((1, N), jnp.bfloat16),
    )(u_bf, vs_bf)

    row2d, idx1 = pl.pallas_call(
        _kb_body,
        grid=(GRID,),
        in_specs=[
            pl.BlockSpec((1, N), lambda j: (0, 0)),
            pl.BlockSpec((N, D), lambda j: (0, 0)),
            pl.BlockSpec((BLK, D), lambda j: (j, 0)),
        ],
        out_specs=[
            pl.BlockSpec((1, BLK), lambda j: (0, j)),
            pl.BlockSpec(memory_space=pltpu.SMEM, block_shape=(1,),
                         index_map=lambda j: (0,)),
        ],
        out_shape=[
            jax.ShapeDtypeStruct((1, N), jnp.float32),
            jax.ShapeDtypeStruct((1,), jnp.int32),
        ],
        scratch_shapes=[
            pltpu.SMEM((1,), jnp.float32),
            pltpu.SMEM((1,), jnp.int32),
        ],
    )(v, t_bf, t_bf)

    row = row2d.reshape(N)
    return (row, idx1[0].astype(jnp.int32))


# speedup vs baseline: 1.1479x; 1.1479x over previous
import functools

import jax
import jax.numpy as jnp
from jax.experimental import pallas as pl
from jax.experimental.pallas import tpu as pltpu

N = 4096
D = 256
BLK = 256
GRID = N // BLK


def _bitonic_sort_lanes(x, n):
    lane = jax.lax.broadcasted_iota(jnp.int32, (1, n), 1)
    k = 2
    while k <= n:
        j = k // 2
        while j >= 1:
            up = jnp.roll(x, -j, axis=1)
            down = jnp.roll(x, j, axis=1)
            low_half = (lane & j) == 0
            partner = jnp.where(low_half, up, down)
            want_min = low_half ^ ((lane & k) != 0)
            x = jnp.where(want_min, jnp.minimum(x, partner),
                          jnp.maximum(x, partner))
            j //= 2
        k *= 2
    return x


def _ka_body(u_ref, vs_ref, v_ref):
    s = jax.lax.dot_general(u_ref[...], vs_ref[...],
                            (((1,), (1,)), ((), ())),
                            preferred_element_type=jnp.float32)
    ssort = _bitonic_sort_lanes(s, N)
    norm2 = jnp.sum(ssort * ssort)
    v_ref[...] = (ssort / jnp.sqrt(norm2)).astype(jnp.bfloat16)


def _bitonic_sort_sublanes(x, n):
    row = jax.lax.broadcasted_iota(jnp.int32, (n, 1), 0)
    strides = [1 << b for b in range(12)]
    jmask = {j: (row & j) == 0 for j in strides}
    kmask = {k: (row & k) != 0 for k in [s * 2 for s in strides]}
    k = 2
    while k <= n:
        j = k // 2
        while j >= 1:
            up = jnp.roll(x, -j, axis=0)
            down = jnp.roll(x, j, axis=0)
            low_half = jmask[j]
            partner = jnp.where(low_half, up, down)
            want_min = low_half ^ kmask[k]
            x = jnp.where(want_min, jnp.minimum(x, partner),
                          jnp.maximum(x, partner))
            j //= 2
        k *= 2
    return x


def _kb_body(v_ref, t_ref, tb_ref, row_ref, idx_ref, maxv_sc, maxi_sc):
    jidx = pl.program_id(0)
    g = jax.lax.dot_general(t_ref[...], tb_ref[...],
                            (((1,), (1,)), ((), ())),
                            preferred_element_type=jnp.float32)
    gs = _bitonic_sort_sublanes(g, N)
    norms2 = jnp.sum(gs * gs, axis=0, keepdims=True)
    gsn = (gs / jnp.sqrt(norms2)).astype(jnp.bfloat16)
    vals = jax.lax.dot_general(v_ref[...], gsn,
                               (((1,), (0,)), ((), ())),
                               preferred_element_type=jnp.float32)
    row_ref[...] = vals

    m = jnp.max(vals)
    lane = jax.lax.broadcasted_iota(jnp.int32, (1, BLK), 1) + jidx * BLK
    li = jnp.min(jnp.where(vals == m, lane, jnp.int32(2**30)))

    prev_v = maxv_sc[0]
    better = jnp.logical_or(jidx == 0, m > prev_v)
    maxv_sc[0] = jnp.where(better, m, prev_v)
    maxi_sc[0] = jnp.where(better, li, maxi_sc[0])

    @pl.when(jidx == GRID - 1)
    def _():
        idx_ref[0] = maxi_sc[0]


def kernel(vectors_source, vectors_target, src_idx):
    u = jnp.take(vectors_source, src_idx, axis=0).reshape(1, D)
    u_bf = u.astype(jnp.bfloat16)
    vs_bf = vectors_source.astype(jnp.bfloat16)
    t_bf = vectors_target.astype(jnp.bfloat16)

    v = pl.pallas_call(
        _ka_body,
        out_shape=jax.ShapeDtypeStruct((1, N), jnp.bfloat16),
    )(u_bf, vs_bf)

    row2d, idx1 = pl.pallas_call(
        _kb_body,
        grid=(GRID,),
        in_specs=[
            pl.BlockSpec((1, N), lambda j: (0, 0)),
            pl.BlockSpec((N, D), lambda j: (0, 0)),
            pl.BlockSpec((BLK, D), lambda j: (j, 0)),
        ],
        out_specs=[
            pl.BlockSpec((1, BLK), lambda j: (0, j)),
            pl.BlockSpec(memory_space=pltpu.SMEM, block_shape=(1,),
                         index_map=lambda j: (0,)),
        ],
        out_shape=[
            jax.ShapeDtypeStruct((1, N), jnp.float32),
            jax.ShapeDtypeStruct((1,), jnp.int32),
        ],
        scratch_shapes=[
            pltpu.SMEM((1,), jnp.float32),
            pltpu.SMEM((1,), jnp.int32),
        ],
    )(v, t_bf, t_bf)

    row = row2d.reshape(N)
    return (row, idx1[0].astype(jnp.int32))


# speedup vs baseline: 1.2399x; 1.0802x over previous
import functools

import jax
import jax.numpy as jnp
from jax.experimental import pallas as pl
from jax.experimental.pallas import tpu as pltpu

N = 4096
D = 256
BLK = 512
GRID = N // BLK


def _bitonic_sort_lanes(x, n):
    lane = jax.lax.broadcasted_iota(jnp.int32, (1, n), 1)
    k = 2
    while k <= n:
        j = k // 2
        while j >= 1:
            up = jnp.roll(x, -j, axis=1)
            down = jnp.roll(x, j, axis=1)
            low_half = (lane & j) == 0
            partner = jnp.where(low_half, up, down)
            want_min = low_half ^ ((lane & k) != 0)
            x = jnp.where(want_min, jnp.minimum(x, partner),
                          jnp.maximum(x, partner))
            j //= 2
        k *= 2
    return x


def _ka_body(u_ref, vs_ref, v_ref):
    s = jax.lax.dot_general(u_ref[...], vs_ref[...],
                            (((1,), (1,)), ((), ())),
                            preferred_element_type=jnp.float32)
    ssort = _bitonic_sort_lanes(s, N)
    norm2 = jnp.sum(ssort * ssort)
    v_ref[...] = (ssort / jnp.sqrt(norm2)).astype(jnp.bfloat16)


def _bitonic_sort_sublanes(x, n):
    row = jax.lax.broadcasted_iota(jnp.int32, (n, 1), 0)
    strides = [1 << b for b in range(12)]
    jmask = {j: (row & j) == 0 for j in strides}
    kmask = {k: (row & k) != 0 for k in [s * 2 for s in strides]}
    k = 2
    while k <= n:
        j = k // 2
        while j >= 1:
            up = jnp.roll(x, -j, axis=0)
            down = jnp.roll(x, j, axis=0)
            low_half = jmask[j]
            partner = jnp.where(low_half, up, down)
            want_min = low_half ^ kmask[k]
            x = jnp.where(want_min, jnp.minimum(x, partner),
                          jnp.maximum(x, partner))
            j //= 2
        k *= 2
    return x


def _kb_body(v_ref, t_ref, tb_ref, row_ref, idx_ref, maxv_sc, maxi_sc):
    jidx = pl.program_id(0)
    g = jax.lax.dot_general(t_ref[...], tb_ref[...],
                            (((1,), (1,)), ((), ())),
                            preferred_element_type=jnp.float32)
    gs = _bitonic_sort_sublanes(g, N)
    norms2 = jnp.sum(gs * gs, axis=0, keepdims=True)
    gsn = (gs / jnp.sqrt(norms2)).astype(jnp.bfloat16)
    vals = jax.lax.dot_general(v_ref[...], gsn,
                               (((1,), (0,)), ((), ())),
                               preferred_element_type=jnp.float32)
    row_ref[...] = vals

    m = jnp.max(vals)
    lane = jax.lax.broadcasted_iota(jnp.int32, (1, BLK), 1) + jidx * BLK
    li = jnp.min(jnp.where(vals == m, lane, jnp.int32(2**30)))

    prev_v = maxv_sc[0]
    better = jnp.logical_or(jidx == 0, m > prev_v)
    maxv_sc[0] = jnp.where(better, m, prev_v)
    maxi_sc[0] = jnp.where(better, li, maxi_sc[0])

    @pl.when(jidx == GRID - 1)
    def _():
        idx_ref[0] = maxi_sc[0]


def kernel(vectors_source, vectors_target, src_idx):
    u = jnp.take(vectors_source, src_idx, axis=0).reshape(1, D)
    u_bf = u.astype(jnp.bfloat16)
    vs_bf = vectors_source.astype(jnp.bfloat16)
    t_bf = vectors_target.astype(jnp.bfloat16)

    v = pl.pallas_call(
        _ka_body,
        out_shape=jax.ShapeDtypeStruct((1, N), jnp.bfloat16),
    )(u_bf, vs_bf)

    row2d, idx1 = pl.pallas_call(
        _kb_body,
        grid=(GRID,),
        in_specs=[
            pl.BlockSpec((1, N), lambda j: (0, 0)),
            pl.BlockSpec((N, D), lambda j: (0, 0)),
            pl.BlockSpec((BLK, D), lambda j: (j, 0)),
        ],
        out_specs=[
            pl.BlockSpec((1, BLK), lambda j: (0, j)),
            pl.BlockSpec(memory_space=pltpu.SMEM, block_shape=(1,),
                         index_map=lambda j: (0,)),
        ],
        out_shape=[
            jax.ShapeDtypeStruct((1, N), jnp.float32),
            jax.ShapeDtypeStruct((1,), jnp.int32),
        ],
        scratch_shapes=[
            pltpu.SMEM((1,), jnp.float32),
            pltpu.SMEM((1,), jnp.int32),
        ],
    )(v, t_bf, t_bf)

    row = row2d.reshape(N)
    return (row, idx1[0].astype(jnp.int32))


# speedup vs baseline: 1.3072x; 1.0543x over previous
import functools

import jax
import jax.numpy as jnp
from jax.experimental import pallas as pl
from jax.experimental.pallas import tpu as pltpu

N = 4096
D = 256
BLK = 1024
GRID = N // BLK


def _bitonic_sort_lanes(x, n):
    lane = jax.lax.broadcasted_iota(jnp.int32, (1, n), 1)
    k = 2
    while k <= n:
        j = k // 2
        while j >= 1:
            up = jnp.roll(x, -j, axis=1)
            down = jnp.roll(x, j, axis=1)
            low_half = (lane & j) == 0
            partner = jnp.where(low_half, up, down)
            want_min = low_half ^ ((lane & k) != 0)
            x = jnp.where(want_min, jnp.minimum(x, partner),
                          jnp.maximum(x, partner))
            j //= 2
        k *= 2
    return x


def _ka_body(u_ref, vs_ref, v_ref):
    s = jax.lax.dot_general(u_ref[...], vs_ref[...],
                            (((1,), (1,)), ((), ())),
                            preferred_element_type=jnp.float32)
    ssort = _bitonic_sort_lanes(s, N)
    norm2 = jnp.sum(ssort * ssort)
    v_ref[...] = (ssort / jnp.sqrt(norm2)).astype(jnp.bfloat16)


def _bitonic_sort_sublanes(x, n):
    row = jax.lax.broadcasted_iota(jnp.int32, (n, 1), 0)
    strides = [1 << b for b in range(12)]
    jmask = {j: (row & j) == 0 for j in strides}
    kmask = {k: (row & k) != 0 for k in [s * 2 for s in strides]}
    k = 2
    while k <= n:
        j = k // 2
        while j >= 1:
            up = jnp.roll(x, -j, axis=0)
            down = jnp.roll(x, j, axis=0)
            low_half = jmask[j]
            partner = jnp.where(low_half, up, down)
            want_min = low_half ^ kmask[k]
            x = jnp.where(want_min, jnp.minimum(x, partner),
                          jnp.maximum(x, partner))
            j //= 2
        k *= 2
    return x


def _kb_body(v_ref, t_ref, tb_ref, row_ref, idx_ref, maxv_sc, maxi_sc):
    jidx = pl.program_id(0)
    g = jax.lax.dot_general(t_ref[...], tb_ref[...],
                            (((1,), (1,)), ((), ())),
                            preferred_element_type=jnp.float32)
    gs = _bitonic_sort_sublanes(g, N)
    norms2 = jnp.sum(gs * gs, axis=0, keepdims=True)
    gsn = (gs / jnp.sqrt(norms2)).astype(jnp.bfloat16)
    vals = jax.lax.dot_general(v_ref[...], gsn,
                               (((1,), (0,)), ((), ())),
                               preferred_element_type=jnp.float32)
    row_ref[...] = vals

    m = jnp.max(vals)
    lane = jax.lax.broadcasted_iota(jnp.int32, (1, BLK), 1) + jidx * BLK
    li = jnp.min(jnp.where(vals == m, lane, jnp.int32(2**30)))

    prev_v = maxv_sc[0]
    better = jnp.logical_or(jidx == 0, m > prev_v)
    maxv_sc[0] = jnp.where(better, m, prev_v)
    maxi_sc[0] = jnp.where(better, li, maxi_sc[0])

    @pl.when(jidx == GRID - 1)
    def _():
        idx_ref[0] = maxi_sc[0]


def kernel(vectors_source, vectors_target, src_idx):
    u = jnp.take(vectors_source, src_idx, axis=0).reshape(1, D)
    u_bf = u.astype(jnp.bfloat16)
    vs_bf = vectors_source.astype(jnp.bfloat16)
    t_bf = vectors_target.astype(jnp.bfloat16)

    v = pl.pallas_call(
        _ka_body,
        out_shape=jax.ShapeDtypeStruct((1, N), jnp.bfloat16),
    )(u_bf, vs_bf)

    row2d, idx1 = pl.pallas_call(
        _kb_body,
        grid=(GRID,),
        in_specs=[
            pl.BlockSpec((1, N), lambda j: (0, 0)),
            pl.BlockSpec((N, D), lambda j: (0, 0)),
            pl.BlockSpec((BLK, D), lambda j: (j, 0)),
        ],
        out_specs=[
            pl.BlockSpec((1, BLK), lambda j: (0, j)),
            pl.BlockSpec(memory_space=pltpu.SMEM, block_shape=(1,),
                         index_map=lambda j: (0,)),
        ],
        out_shape=[
            jax.ShapeDtypeStruct((1, N), jnp.float32),
            jax.ShapeDtypeStruct((1,), jnp.int32),
        ],
        scratch_shapes=[
            pltpu.SMEM((1,), jnp.float32),
            pltpu.SMEM((1,), jnp.int32),
        ],
    )(v, t_bf, t_bf)

    row = row2d.reshape(N)
    return (row, idx1[0].astype(jnp.int32))
